# baseline (speedup 1.0000x reference)
import functools
import math

import jax
import jax.numpy as jnp
from jax import lax
from jax.experimental import pallas as pl
from jax.experimental.pallas import tpu as pltpu
from jax.experimental.pallas import tpu_sc as plsc

_N = 10000
_H = 128
_E = 320000
_L = 3
_BN_EPS = 1e-5

_NC = 2
_NS = 16
_NW = _NC * _NS
_CHUNK = 80
_KCH = 128
_NBUF = 4
_LEAD = 3
_GB = 8
_NCHT = _E // _CHUNK
_NBL = _KCH // _GB
_NBS = (_NCHT - 31 * _KCH) // _GB
_NPAD = 10240
_RPT = _NPAD // _NS


def _sc_agg_body(h_hbm, sd_hbm, zeros_hbm, out_hbm,
                 src_v, dst_v, rows_v, acc,
                 isem, gsem0, gsem1, gsem2, gsem3, ssem0, ssem1, ssem2, ssem3):
    gsems = (gsem0, gsem1, gsem2, gsem3)
    ssems = (ssem0, ssem1, ssem2, ssem3)
    cid = lax.axis_index("c")
    sid = lax.axis_index("s")
    wid = cid * _NS + sid
    base = wid * _NBL
    nb = jnp.where(wid == _NW - 1, _NBS, _NBL)
    zc = pltpu.make_async_copy(zeros_hbm.at[pl.ds(sid * _RPT, _RPT)],
                               acc.at[pl.ds(sid * _RPT, _RPT)], ssems[-1])
    zc.start()
    i0s = pltpu.make_async_copy(sd_hbm.at[0, base], src_v.at[0], ssems[0])
    i0d = pltpu.make_async_copy(sd_hbm.at[1, base], dst_v.at[0], ssems[0])
    i0s.start()
    i0d.start()
    i0s.wait()
    i0d.wait()
    zc.wait()
    plsc.subcore_barrier()

    _sc_edge_loop(h_hbm, sd_hbm, src_v, dst_v, rows_v, acc,
                  base, nb, isem, gsems, ssems)
    plsc.subcore_barrier()
    pltpu.sync_copy(acc.at[pl.ds(sid * _RPT, _RPT)],
                    out_hbm.at[cid, pl.ds(sid * _RPT, _RPT)])


def _sc_edge_loop(h_hbm, sd_hbm, src_v, dst_v, rows_v, acc,
                  base, nb, isem, gsems, ssems):
    pltpu.async_copy(sd_hbm.at[0, base + 1], src_v.at[1], isem)
    pltpu.async_copy(sd_hbm.at[1, base + 1], dst_v.at[1], isem)
    for b in range(_LEAD):
        pltpu.async_copy(h_hbm.at[src_v.at[0, b]], rows_v.at[b], gsems[b])

    def _drain_idx():
        pltpu.make_async_copy(sd_hbm.at[0, base], src_v.at[0],
                              isem).wait()
        pltpu.make_async_copy(sd_hbm.at[1, base], dst_v.at[0],
                              isem).wait()

    def batch(m, carry):
        s = lax.rem(m, 2)
        for q in range(_GB):
            b = q % _NBUF
            bn = (q + _LEAD) % _NBUF
            j = m * _GB + q
            @pl.when(j >= _NBUF - _LEAD)
            def _():
                pltpu.make_async_copy(
                    rows_v.at[bn], acc.at[dst_v.at[s, q]], ssems[bn]).wait()
            if q == 2:
                @pl.when(jnp.logical_and(m > 0, m + 1 < nb))
                def _():
                    off = base + m + 1
                    pltpu.async_copy(sd_hbm.at[0, off],
                                     src_v.at[1 - s], isem)
                    pltpu.async_copy(sd_hbm.at[1, off],
                                     dst_v.at[1 - s], isem)
            if q == _GB - _LEAD:
                @pl.when(m + 1 < nb)
                def _():
                    _drain_idx()
            if q < _GB - _LEAD:
                pltpu.async_copy(h_hbm.at[src_v.at[s, q + _LEAD]],
                                 rows_v.at[bn], gsems[bn])
            else:
                @pl.when(m + 1 < nb)
                def _():
                    pltpu.async_copy(
                        h_hbm.at[src_v.at[1 - s, q + _LEAD - _GB]],
                        rows_v.at[bn], gsems[bn])
            pltpu.make_async_copy(h_hbm.at[src_v.at[s, q]], rows_v.at[b],
                                  gsems[b]).wait()
            pltpu.make_async_copy(rows_v.at[b], acc.at[dst_v.at[s, q]],
                                  ssems[b]).start(add=True)
        return carry

    lax.fori_loop(0, nb, batch, 0)
    for k in range(_NBUF - _LEAD, 0, -1):
        pltpu.make_async_copy(rows_v.at[(_GB - k) % _NBUF],
                              acc.at[dst_v.at[1, _GB - k]],
                              ssems[(_GB - k) % _NBUF]).wait()


_sc_agg = functools.partial(
    pl.kernel,
    out_type=jax.ShapeDtypeStruct((_NC, _NPAD, _H), jnp.float32),
    mesh=plsc.VectorSubcoreMesh(core_axis_name="c", subcore_axis_name="s"),
    scratch_types=[
        pltpu.VMEM((2, _GB, _CHUNK), jnp.int32),
        pltpu.VMEM((2, _GB, _CHUNK), jnp.int32),
        pltpu.VMEM((_NBUF, _CHUNK, _H), jnp.float32),
        pltpu.VMEM_SHARED((_NPAD, _H), jnp.float32),
    ] + [pltpu.SemaphoreType.DMA] * 9,
)(_sc_agg_body)


_ROWBLK = 2048
_BN_INV = 1.0 / math.sqrt(1.0 + _BN_EPS)


def _mlp_body(x_ref, a0_ref, a1_ref, w1_ref, b1_ref, g_ref, bt_ref,
              w2_ref, b2_ref, o_ref):
    h = x_ref[...] + a0_ref[0] + a1_ref[0]
    t = jnp.dot(h, w1_ref[...], preferred_element_type=jnp.float32)
    t = (t + b1_ref[...]) * (g_ref[...] * _BN_INV) + bt_ref[...]
    t = jnp.maximum(t, 0.0)
    u = jnp.dot(t, w2_ref[...], preferred_element_type=jnp.float32)
    o_ref[...] = jnp.maximum(u + b2_ref[...], 0.0)


def _mlp(x, aggs, w1, b1, g, bt, w2, b2):
    grid = (_N + _ROWBLK - 1) // _ROWBLK
    row_spec = pl.BlockSpec((_ROWBLK, _H), lambda i: (i, 0))
    a0_spec = pl.BlockSpec((1, _ROWBLK, _H), lambda i: (0, i, 0))
    a1_spec = pl.BlockSpec((1, _ROWBLK, _H), lambda i: (1, i, 0))
    full_spec = pl.BlockSpec((_H, _H), lambda i: (0, 0))
    vec_spec = pl.BlockSpec((1, _H), lambda i: (0, 0))
    return pl.pallas_call(
        _mlp_body,
        grid=(grid,),
        in_specs=[row_spec, a0_spec, a1_spec,
                  full_spec, vec_spec, vec_spec, vec_spec,
                  full_spec, vec_spec],
        out_specs=row_spec,
        out_shape=jax.ShapeDtypeStruct((_N, _H), jnp.float32),
    )(x, aggs, aggs, w1, b1.reshape(1, _H), g.reshape(1, _H),
      bt.reshape(1, _H), w2, b2.reshape(1, _H))


def kernel(x, adj_t,
           W1_0, b1_0, g_0, bt_0, W2_0, b2_0,
           W1_1, b1_1, g_1, bt_1, W2_1, b2_1,
           W1_2, b1_2, g_2, bt_2, W2_2, b2_2):
    sd = adj_t.reshape(2, _NCHT // _GB, _GB, _CHUNK)
    zeros = jnp.zeros((_NPAD, _H), jnp.float32)

    params = [
        (W1_0, b1_0, g_0, bt_0, W2_0, b2_0),
        (W1_1, b1_1, g_1, bt_1, W2_1, b2_1),
        (W1_2, b1_2, g_2, bt_2, W2_2, b2_2),
    ]
    h = x
    for (w1, b1, g, bt, w2, b2) in params:
        aggs = _sc_agg(h, sd, zeros)
        h = _mlp(h, aggs, w1, b1, g, bt, w2, b2)
    return h


# speedup vs baseline: 1.0030x; 1.0030x over previous
import functools
import math

import jax
import jax.numpy as jnp
from jax import lax
from jax.experimental import pallas as pl
from jax.experimental.pallas import tpu as pltpu
from jax.experimental.pallas import tpu_sc as plsc

_N = 10000
_H = 128
_E = 320000
_L = 3
_BN_EPS = 1e-5

_NC = 2
_NS = 16
_NW = _NC * _NS
_CHUNK = 80
_KCH = 128
_NBUF = 4
_LEAD = 3
_GB = 8
_NCHT = _E // _CHUNK
_NBL = _KCH // _GB
_NBS = (_NCHT - 31 * _KCH) // _GB
_NPAD = 10112
_RPT = _NPAD // _NS


def _sc_agg_body(h_hbm, sd_hbm, zeros_hbm, out_hbm,
                 src_v, dst_v, rows_v, acc,
                 isem, gsem0, gsem1, gsem2, gsem3, ssem0, ssem1, ssem2, ssem3):
    gsems = (gsem0, gsem1, gsem2, gsem3)
    ssems = (ssem0, ssem1, ssem2, ssem3)
    cid = lax.axis_index("c")
    sid = lax.axis_index("s")
    wid = cid * _NS + sid
    base = wid * _NBL
    nb = jnp.where(wid == _NW - 1, _NBS, _NBL)
    zc = pltpu.make_async_copy(zeros_hbm.at[pl.ds(sid * _RPT, _RPT)],
                               acc.at[pl.ds(sid * _RPT, _RPT)], ssems[-1])
    zc.start()
    i0s = pltpu.make_async_copy(sd_hbm.at[0, base], src_v.at[0], ssems[0])
    i0d = pltpu.make_async_copy(sd_hbm.at[1, base], dst_v.at[0], ssems[0])
    i0s.start()
    i0d.start()
    i0s.wait()
    i0d.wait()
    zc.wait()
    plsc.subcore_barrier()

    _sc_edge_loop(h_hbm, sd_hbm, src_v, dst_v, rows_v, acc,
                  base, nb, isem, gsems, ssems)
    plsc.subcore_barrier()
    pltpu.sync_copy(acc.at[pl.ds(sid * _RPT, _RPT)],
                    out_hbm.at[cid, pl.ds(sid * _RPT, _RPT)])


def _sc_edge_loop(h_hbm, sd_hbm, src_v, dst_v, rows_v, acc,
                  base, nb, isem, gsems, ssems):
    pltpu.async_copy(sd_hbm.at[0, base + 1], src_v.at[1], isem)
    pltpu.async_copy(sd_hbm.at[1, base + 1], dst_v.at[1], isem)
    for b in range(_LEAD):
        pltpu.async_copy(h_hbm.at[src_v.at[0, b]], rows_v.at[b], gsems[b])

    def _drain_idx():
        pltpu.make_async_copy(sd_hbm.at[0, base], src_v.at[0],
                              isem).wait()
        pltpu.make_async_copy(sd_hbm.at[1, base], dst_v.at[0],
                              isem).wait()

    def batch(m, carry):
        s = lax.rem(m, 2)
        for q in range(_GB):
            b = q % _NBUF
            bn = (q + _LEAD) % _NBUF
            j = m * _GB + q
            @pl.when(j >= _NBUF - _LEAD)
            def _():
                pltpu.make_async_copy(
                    rows_v.at[bn], acc.at[dst_v.at[s, q]], ssems[bn]).wait()
            if q == 2:
                @pl.when(jnp.logical_and(m > 0, m + 1 < nb))
                def _():
                    off = base + m + 1
                    pltpu.async_copy(sd_hbm.at[0, off],
                                     src_v.at[1 - s], isem)
                    pltpu.async_copy(sd_hbm.at[1, off],
                                     dst_v.at[1 - s], isem)
            if q == _GB - _LEAD:
                @pl.when(m + 1 < nb)
                def _():
                    _drain_idx()
            if q < _GB - _LEAD:
                pltpu.async_copy(h_hbm.at[src_v.at[s, q + _LEAD]],
                                 rows_v.at[bn], gsems[bn])
            else:
                @pl.when(m + 1 < nb)
                def _():
                    pltpu.async_copy(
                        h_hbm.at[src_v.at[1 - s, q + _LEAD - _GB]],
                        rows_v.at[bn], gsems[bn])
            pltpu.make_async_copy(h_hbm.at[src_v.at[s, q]], rows_v.at[b],
                                  gsems[b]).wait()
            pltpu.make_async_copy(rows_v.at[b], acc.at[dst_v.at[s, q]],
                                  ssems[b]).start(add=True)
        return carry

    lax.fori_loop(0, nb, batch, 0)
    for k in range(_NBUF - _LEAD, 0, -1):
        pltpu.make_async_copy(rows_v.at[(_GB - k) % _NBUF],
                              acc.at[dst_v.at[1, _GB - k]],
                              ssems[(_GB - k) % _NBUF]).wait()


_sc_agg = functools.partial(
    pl.kernel,
    out_type=jax.ShapeDtypeStruct((_NC, _NPAD, _H), jnp.float32),
    mesh=plsc.VectorSubcoreMesh(core_axis_name="c", subcore_axis_name="s"),
    scratch_types=[
        pltpu.VMEM((2, _GB, _CHUNK), jnp.int32),
        pltpu.VMEM((2, _GB, _CHUNK), jnp.int32),
        pltpu.VMEM((_NBUF, _CHUNK, _H), jnp.float32),
        pltpu.VMEM_SHARED((_NPAD, _H), jnp.float32),
    ] + [pltpu.SemaphoreType.DMA] * 9,
)(_sc_agg_body)


_ROWBLK = 2048
_BN_INV = 1.0 / math.sqrt(1.0 + _BN_EPS)


def _mlp_body(x_ref, a0_ref, a1_ref, w1_ref, b1_ref, g_ref, bt_ref,
              w2_ref, b2_ref, o_ref):
    h = x_ref[...] + a0_ref[0] + a1_ref[0]
    t = jnp.dot(h, w1_ref[...], preferred_element_type=jnp.float32)
    t = (t + b1_ref[...]) * (g_ref[...] * _BN_INV) + bt_ref[...]
    t = jnp.maximum(t, 0.0)
    u = jnp.dot(t, w2_ref[...], preferred_element_type=jnp.float32)
    o_ref[...] = jnp.maximum(u + b2_ref[...], 0.0)


def _mlp(x, aggs, w1, b1, g, bt, w2, b2):
    grid = (_N + _ROWBLK - 1) // _ROWBLK
    row_spec = pl.BlockSpec((_ROWBLK, _H), lambda i: (i, 0))
    a0_spec = pl.BlockSpec((1, _ROWBLK, _H), lambda i: (0, i, 0))
    a1_spec = pl.BlockSpec((1, _ROWBLK, _H), lambda i: (1, i, 0))
    full_spec = pl.BlockSpec((_H, _H), lambda i: (0, 0))
    vec_spec = pl.BlockSpec((1, _H), lambda i: (0, 0))
    return pl.pallas_call(
        _mlp_body,
        grid=(grid,),
        in_specs=[row_spec, a0_spec, a1_spec,
                  full_spec, vec_spec, vec_spec, vec_spec,
                  full_spec, vec_spec],
        out_specs=row_spec,
        out_shape=jax.ShapeDtypeStruct((_N, _H), jnp.float32),
    )(x, aggs, aggs, w1, b1.reshape(1, _H), g.reshape(1, _H),
      bt.reshape(1, _H), w2, b2.reshape(1, _H))


def kernel(x, adj_t,
           W1_0, b1_0, g_0, bt_0, W2_0, b2_0,
           W1_1, b1_1, g_1, bt_1, W2_1, b2_1,
           W1_2, b1_2, g_2, bt_2, W2_2, b2_2):
    sd = adj_t.reshape(2, _NCHT // _GB, _GB, _CHUNK)
    zeros = jnp.zeros((_NPAD, _H), jnp.float32)

    params = [
        (W1_0, b1_0, g_0, bt_0, W2_0, b2_0),
        (W1_1, b1_1, g_1, bt_1, W2_1, b2_1),
        (W1_2, b1_2, g_2, bt_2, W2_2, b2_2),
    ]
    h = x
    for (w1, b1, g, bt, w2, b2) in params:
        aggs = _sc_agg(h, sd, zeros)
        h = _mlp(h, aggs, w1, b1, g, bt, w2, b2)
    return h
